# baseline (speedup 1.0000x reference)
import functools

import jax
import jax.numpy as jnp
from jax import lax
from jax.experimental import pallas as pl
from jax.experimental.pallas import tpu as pltpu
from jax.experimental.pallas import tpu_sc as plsc

B, T, V, E = 4096, 200, 1000000, 32
H1, H2, H3 = 500, 100, 1
N = B * T
NCH = (T * E) // 128

NC, NS = 2, 16
NW = NC * NS

B_PER_W = N // NW
CHUNK = 128
N_CHUNKS = B_PER_W // CHUNK


NBLK = V // 128
TAIL = V - NBLK * 128


def _idx_row(idx_v, r):
    return idx_v[r // 8, pl.ds(16 * (r % 8), 16)]


def _tpose_block(src, dst, idx_v, nk):
    for e in range(32):
        cols = _idx_row(idx_v, 8 + e)
        for k in range(nk):
            vec = src[e, pl.ds(16 * k, 16)]
            plsc.store_scatter(dst, [_idx_row(idx_v, k), cols], vec)


def _idx_tab():
    iota = jnp.arange(16, dtype=jnp.int32)
    rows = 4 * jnp.arange(8, dtype=jnp.int32)[:, None] + iota[None, :] // 4
    cols = 32 * (iota[None, :] % 4) + jnp.arange(32, dtype=jnp.int32)[:, None]
    pad = jnp.zeros((24, 16), jnp.int32)
    return jnp.concatenate([rows, cols, pad], axis=0).reshape(8, 128)


def _fmt_sc_body(embt_hbm, tail_hbm, idxtab_hbm, tbl_hbm, in0, in1, out0,
                 out1, idx_v, is0, is1, os0, os1):
    wid = lax.axis_index("s") * NC + lax.axis_index("c")
    pltpu.sync_copy(idxtab_hbm, idx_v)
    start = wid * 244 + jnp.minimum(wid, 4)
    count = jnp.where(wid < 4, 245, 244)
    npairs = (count + 1) // 2

    def in_copy(buf, c):
        return pltpu.make_async_copy(
            embt_hbm.at[:, pl.ds(c * 128, 128)], buf,
            is0 if buf is in0 else is1,
        )

    def out_copy(buf, c):
        return pltpu.make_async_copy(
            buf, tbl_hbm.at[pl.ds(c * 32, 32)],
            os0 if buf is out0 else os1,
        )

    in_copy(in0, start).start()
    in_copy(in1, start + 1).start()

    def pair(j, _):
        c0 = start + 2 * j
        c1 = c0 + 1
        in_copy(in0, c0).wait()

        @pl.when(j >= 1)
        def _():
            out_copy(out0, c0).wait()

        _tpose_block(in0, out0, idx_v, 8)
        out_copy(out0, c0).start()

        @pl.when(2 * j + 2 < count)
        def _():
            in_copy(in0, c0 + 2).start()

        @pl.when(2 * j + 1 < count)
        def _():
            in_copy(in1, c1).wait()

            @pl.when(j >= 1)
            def _():
                out_copy(out1, c1).wait()

            _tpose_block(in1, out1, idx_v, 8)
            out_copy(out1, c1).start()

            @pl.when(2 * j + 3 < count)
            def _():
                in_copy(in1, c1 + 2).start()

        return ()

    lax.fori_loop(0, npairs, pair, ())
    out_copy(out0, 0).wait()
    out_copy(out1, 0).wait()

    @pl.when(wid == NW - 1)
    def _tail():
        t = pltpu.make_async_copy(tail_hbm, in0.at[pl.ds(0, TAIL // 4)], is0)
        t.start()
        t.wait()
        t2 = pltpu.make_async_copy(
            in0.at[pl.ds(0, TAIL // 4)],
            tbl_hbm.at[pl.ds(NBLK * 32, TAIL // 4)], os0,
        )
        t2.start()
        t2.wait()


@functools.cache
def _fmt_sc():
    mesh = plsc.VectorSubcoreMesh(
        core_axis_name="c", subcore_axis_name="s", num_cores=NC
    )
    return pl.kernel(
        _fmt_sc_body,
        mesh=mesh,
        out_type=jax.ShapeDtypeStruct((V // 4, 128), jnp.float32),
        scratch_types=[
            pltpu.VMEM((32, 128), jnp.float32),
            pltpu.VMEM((32, 128), jnp.float32),
            pltpu.VMEM((32, 128), jnp.float32),
            pltpu.VMEM((32, 128), jnp.float32),
            pltpu.VMEM((8, 128), jnp.int32),
            pltpu.SemaphoreType.DMA,
            pltpu.SemaphoreType.DMA,
            pltpu.SemaphoreType.DMA,
            pltpu.SemaphoreType.DMA,
        ],
        compiler_params=pltpu.CompilerParams(
            use_tc_tiling_on_sc=True, needs_layout_passes=False
        ),
    )


def _gather_sc_body(idx_hbm, dst_hbm, table_hbm, out_hbm,
                    idx_v, dst_v, rows_v, gsem, wsem):
    wid = lax.axis_index("s") * NC + lax.axis_index("c")
    pltpu.sync_copy(idx_hbm.at[wid], idx_v)
    pltpu.sync_copy(dst_hbm.at[wid], dst_v)

    def body(j, _):
        buf = lax.rem(j, 2)
        g = pltpu.make_async_copy(table_hbm.at[idx_v.at[j]], rows_v.at[buf], gsem)
        g.start()
        g.wait()
        w = pltpu.make_async_copy(rows_v.at[buf], out_hbm.at[dst_v.at[j]], wsem)
        w.start()
        w.wait()
        return ()

    lax.fori_loop(0, N_CHUNKS, body, ())


@functools.cache
def _gather_sc():
    mesh = plsc.VectorSubcoreMesh(
        core_axis_name="c", subcore_axis_name="s", num_cores=NC
    )
    return pl.kernel(
        _gather_sc_body,
        mesh=mesh,
        out_type=jax.ShapeDtypeStruct((N, E), jnp.float32),
        scratch_types=[
            pltpu.VMEM((N_CHUNKS, CHUNK), jnp.int32),
            pltpu.VMEM((N_CHUNKS, CHUNK), jnp.int32),
            pltpu.VMEM((2, CHUNK, E), jnp.float32),
            pltpu.SemaphoreType.DMA,
            pltpu.SemaphoreType.DMA,
        ],
        compiler_params=pltpu.CompilerParams(use_tc_tiling_on_sc=False),
    )


def _dst_map():
    m = jnp.arange(N, dtype=jnp.int32)
    b = m // T
    t = m - b * T
    p = (t // 4 * B + b) * 4 + (t - t // 4 * 4)
    return p.reshape(NW, N_CHUNKS, CHUNK)


def _mlp_body(h_ref, w1t_ref, b1_ref, w2t_ref, b2_ref, w3_ref, b3_ref,
              o_ref, acc_ref):
    c = pl.program_id(0)

    @pl.when(c == 0)
    def _init():
        acc_ref[...] = jnp.zeros_like(acc_ref)

    acc_ref[...] += lax.dot_general(
        h_ref[0], w1t_ref[...], (((1,), (1,)), ((), ())),
        preferred_element_type=jnp.float32,
    )

    @pl.when(c == NCH - 1)
    def _finish():
        a1 = jnp.maximum(acc_ref[...] + b1_ref[...], 0.0)
        a2 = lax.dot_general(
            a1, w2t_ref[...], (((1,), (1,)), ((), ())),
            preferred_element_type=jnp.float32,
        )
        a2 = jnp.maximum(a2 + b2_ref[...], 0.0)
        a3 = jnp.dot(a2, w3_ref[...], preferred_element_type=jnp.float32)
        o_ref[...] = jax.nn.sigmoid(a3 + b3_ref[...])


def _mlp_tc(h2, W1t, b1, W2t, b2, W3, b3):
    return pl.pallas_call(
        _mlp_body,
        grid=(NCH,),
        in_specs=[
            pl.BlockSpec((1, B, 128), lambda c: (c, 0, 0)),
            pl.BlockSpec((H1, 128), lambda c: (0, c)),
            pl.BlockSpec((1, H1), lambda c: (0, 0)),
            pl.BlockSpec((H2, H1), lambda c: (0, 0)),
            pl.BlockSpec((1, H2), lambda c: (0, 0)),
            pl.BlockSpec((H2, H3), lambda c: (0, 0)),
            pl.BlockSpec((1, H3), lambda c: (0, 0)),
        ],
        out_specs=pl.BlockSpec((B, H3), lambda c: (0, 0)),
        out_shape=jax.ShapeDtypeStruct((B, H3), jnp.float32),
        scratch_shapes=[pltpu.VMEM((B, H1), jnp.float32)],
    )(h2, W1t, b1.reshape(1, H1), W2t, b2.reshape(1, H2), W3,
      b3.reshape(1, H3))


@jax.jit
def kernel(x, emb, W1, b1, W2, b2, W3, b3):
    xm = x.reshape(NW, N_CHUNKS, CHUNK)
    tail = emb[NBLK * 128:, :].reshape(TAIL // 4, 128)
    tbl = _fmt_sc()(emb.T, tail, _idx_tab())
    rows = _gather_sc()(xm, _dst_map(), tbl.reshape(V, E))
    h2 = rows.reshape(NCH, B, 128)
    return _mlp_tc(h2, W1.T, b1, W2.T, b2, W3, b3)


# speedup vs baseline: 1.1336x; 1.1336x over previous
import functools

import jax
import jax.numpy as jnp
from jax import lax
from jax.experimental import pallas as pl
from jax.experimental.pallas import tpu as pltpu
from jax.experimental.pallas import tpu_sc as plsc

B, T, V, E = 4096, 200, 1000000, 32
H1, H2, H3 = 500, 100, 1
N = B * T
NCH = (T * E) // 128

NC, NS = 2, 16
NW = NC * NS

B_PER_W = N // NW
CHUNK = 128
N_CHUNKS = B_PER_W // CHUNK


NBLK = V // 128
TAIL = V - NBLK * 128


def _idx_row(idx_v, r):
    return idx_v[r // 8, pl.ds(16 * (r % 8), 16)]


def _tpose_block(src, dst, idx_v, nk):
    for e in range(32):
        cols = _idx_row(idx_v, 8 + e)
        for k in range(nk):
            vec = src[e, pl.ds(16 * k, 16)]
            plsc.store_scatter(dst, [_idx_row(idx_v, k), cols], vec)


def _idx_tab():
    iota = jnp.arange(16, dtype=jnp.int32)
    rows = 4 * jnp.arange(8, dtype=jnp.int32)[:, None] + iota[None, :] // 4
    cols = 32 * (iota[None, :] % 4) + jnp.arange(32, dtype=jnp.int32)[:, None]
    pad = jnp.zeros((24, 16), jnp.int32)
    return jnp.concatenate([rows, cols, pad], axis=0).reshape(8, 128)


def _fmt_sc_body(embt_hbm, tail_hbm, idxtab_hbm, tbl_hbm, in0, in1, out0,
                 out1, idx_v, is0, is1, os0, os1):
    wid = lax.axis_index("s") * NC + lax.axis_index("c")
    pltpu.sync_copy(idxtab_hbm, idx_v)
    start = wid * 244 + jnp.minimum(wid, 4)
    count = jnp.where(wid < 4, 245, 244)
    npairs = (count + 1) // 2

    def in_copy(buf, c):
        return pltpu.make_async_copy(
            embt_hbm.at[:, pl.ds(c * 128, 128)], buf,
            is0 if buf is in0 else is1,
        )

    def out_copy(buf, c):
        return pltpu.make_async_copy(
            buf, tbl_hbm.at[pl.ds(c * 32, 32)],
            os0 if buf is out0 else os1,
        )

    in_copy(in0, start).start()
    in_copy(in1, start + 1).start()

    def pair(j, _):
        c0 = start + 2 * j
        c1 = c0 + 1
        in_copy(in0, c0).wait()

        @pl.when(j >= 1)
        def _():
            out_copy(out0, c0).wait()

        _tpose_block(in0, out0, idx_v, 8)
        out_copy(out0, c0).start()

        @pl.when(2 * j + 2 < count)
        def _():
            in_copy(in0, c0 + 2).start()

        @pl.when(2 * j + 1 < count)
        def _():
            in_copy(in1, c1).wait()

            @pl.when(j >= 1)
            def _():
                out_copy(out1, c1).wait()

            _tpose_block(in1, out1, idx_v, 8)
            out_copy(out1, c1).start()

            @pl.when(2 * j + 3 < count)
            def _():
                in_copy(in1, c1 + 2).start()

        return ()

    lax.fori_loop(0, npairs, pair, ())
    out_copy(out0, 0).wait()
    out_copy(out1, 0).wait()

    @pl.when(wid == NW - 1)
    def _tail():
        t = pltpu.make_async_copy(tail_hbm, in0.at[pl.ds(0, TAIL // 4)], is0)
        t.start()
        t.wait()
        t2 = pltpu.make_async_copy(
            in0.at[pl.ds(0, TAIL // 4)],
            tbl_hbm.at[pl.ds(NBLK * 32, TAIL // 4)], os0,
        )
        t2.start()
        t2.wait()


@functools.cache
def _fmt_sc():
    mesh = plsc.VectorSubcoreMesh(
        core_axis_name="c", subcore_axis_name="s", num_cores=NC
    )
    return pl.kernel(
        _fmt_sc_body,
        mesh=mesh,
        out_type=jax.ShapeDtypeStruct((V // 4, 128), jnp.float32),
        scratch_types=[
            pltpu.VMEM((32, 128), jnp.float32),
            pltpu.VMEM((32, 128), jnp.float32),
            pltpu.VMEM((32, 128), jnp.float32),
            pltpu.VMEM((32, 128), jnp.float32),
            pltpu.VMEM((8, 128), jnp.int32),
            pltpu.SemaphoreType.DMA,
            pltpu.SemaphoreType.DMA,
            pltpu.SemaphoreType.DMA,
            pltpu.SemaphoreType.DMA,
        ],
        compiler_params=pltpu.CompilerParams(
            use_tc_tiling_on_sc=True, needs_layout_passes=False
        ),
    )


GBUF = 8


def _gather_sc_body(idx_hbm, dst_hbm, table_hbm, out_hbm,
                    idx_v, dst_v, rows_v, gsem, wsem):
    wid = lax.axis_index("s") * NC + lax.axis_index("c")
    pltpu.sync_copy(idx_hbm.at[wid], idx_v)
    pltpu.sync_copy(dst_hbm.at[wid], dst_v)

    def g(b, j):
        return pltpu.make_async_copy(
            table_hbm.at[idx_v.at[j]], rows_v.at[b], gsem
        )

    def s(b, j):
        return pltpu.make_async_copy(
            rows_v.at[b], out_hbm.at[dst_v.at[j]], wsem
        )

    for b in range(GBUF):
        g(b, b).start()

    def group(q, _):
        for b in range(GBUF):
            j = q * GBUF + b
            g(b, j).wait()
            s(b, j).start()
        for b in range(GBUF):
            j = q * GBUF + b
            s(b, j).wait()

            @pl.when(j + GBUF < N_CHUNKS)
            def _():
                g(b, j + GBUF).start()

        return ()

    lax.fori_loop(0, N_CHUNKS // GBUF, group, ())


@functools.cache
def _gather_sc():
    mesh = plsc.VectorSubcoreMesh(
        core_axis_name="c", subcore_axis_name="s", num_cores=NC
    )
    return pl.kernel(
        _gather_sc_body,
        mesh=mesh,
        out_type=jax.ShapeDtypeStruct((N, E), jnp.float32),
        scratch_types=[
            pltpu.VMEM((N_CHUNKS, CHUNK), jnp.int32),
            pltpu.VMEM((N_CHUNKS, CHUNK), jnp.int32),
            pltpu.VMEM((GBUF, CHUNK, E), jnp.float32),
            pltpu.SemaphoreType.DMA,
            pltpu.SemaphoreType.DMA,
        ],
        compiler_params=pltpu.CompilerParams(use_tc_tiling_on_sc=False),
    )


def _dst_map():
    m = jnp.arange(N, dtype=jnp.int32)
    b = m // T
    t = m - b * T
    p = (t // 4 * B + b) * 4 + (t - t // 4 * 4)
    return p.reshape(NW, N_CHUNKS, CHUNK)


def _mlp_body(h_ref, w1t_ref, b1_ref, w2t_ref, b2_ref, w3_ref, b3_ref,
              o_ref, acc_ref):
    c = pl.program_id(0)

    @pl.when(c == 0)
    def _init():
        acc_ref[...] = jnp.zeros_like(acc_ref)

    acc_ref[...] += lax.dot_general(
        h_ref[0], w1t_ref[...], (((1,), (1,)), ((), ())),
        preferred_element_type=jnp.float32,
    )

    @pl.when(c == NCH - 1)
    def _finish():
        a1 = jnp.maximum(acc_ref[...] + b1_ref[...], 0.0)
        a2 = lax.dot_general(
            a1, w2t_ref[...], (((1,), (1,)), ((), ())),
            preferred_element_type=jnp.float32,
        )
        a2 = jnp.maximum(a2 + b2_ref[...], 0.0)
        a3 = jnp.dot(a2, w3_ref[...], preferred_element_type=jnp.float32)
        o_ref[...] = jax.nn.sigmoid(a3 + b3_ref[...])


def _mlp_tc(h2, W1t, b1, W2t, b2, W3, b3):
    return pl.pallas_call(
        _mlp_body,
        grid=(NCH,),
        in_specs=[
            pl.BlockSpec((1, B, 128), lambda c: (c, 0, 0)),
            pl.BlockSpec((H1, 128), lambda c: (0, c)),
            pl.BlockSpec((1, H1), lambda c: (0, 0)),
            pl.BlockSpec((H2, H1), lambda c: (0, 0)),
            pl.BlockSpec((1, H2), lambda c: (0, 0)),
            pl.BlockSpec((H2, H3), lambda c: (0, 0)),
            pl.BlockSpec((1, H3), lambda c: (0, 0)),
        ],
        out_specs=pl.BlockSpec((B, H3), lambda c: (0, 0)),
        out_shape=jax.ShapeDtypeStruct((B, H3), jnp.float32),
        scratch_shapes=[pltpu.VMEM((B, H1), jnp.float32)],
    )(h2, W1t, b1.reshape(1, H1), W2t, b2.reshape(1, H2), W3,
      b3.reshape(1, H3))


@jax.jit
def kernel(x, emb, W1, b1, W2, b2, W3, b3):
    xm = x.reshape(NW, N_CHUNKS, CHUNK)
    tail = emb[NBLK * 128:, :].reshape(TAIL // 4, 128)
    tbl = _fmt_sc()(emb.T, tail, _idx_tab())
    rows = _gather_sc()(xm, _dst_map(), tbl.reshape(V, E))
    h2 = rows.reshape(NCH, B, 128)
    return _mlp_tc(h2, W1.T, b1, W2.T, b2, W3, b3)


# speedup vs baseline: 1.8563x; 1.6376x over previous
import functools

import jax
import jax.numpy as jnp
from jax import lax
from jax.experimental import pallas as pl
from jax.experimental.pallas import tpu as pltpu
from jax.experimental.pallas import tpu_sc as plsc

B, T, V, E = 4096, 200, 1000000, 32
H1, H2, H3 = 500, 100, 1
N = B * T
NCH = (T * E) // 128

NC, NS = 2, 16
NW = NC * NS

B_PER_W = N // NW
CHUNK = 128
N_CHUNKS = B_PER_W // CHUNK


NBLK = V // 128
TAIL = V - NBLK * 128


def _idx_row(idx_v, r):
    return idx_v[r // 8, pl.ds(16 * (r % 8), 16)]


def _tpose_block(src, dst, idx_v, nk):
    for e in range(32):
        cols = _idx_row(idx_v, 8 + e)
        for k in range(nk):
            vec = src[e, pl.ds(16 * k, 16)]
            plsc.store_scatter(dst, [_idx_row(idx_v, k), cols], vec)


def _idx_tab():
    iota = jnp.arange(16, dtype=jnp.int32)
    rows = 4 * jnp.arange(8, dtype=jnp.int32)[:, None] + iota[None, :] // 4
    cols = 32 * (iota[None, :] % 4) + jnp.arange(32, dtype=jnp.int32)[:, None]
    pad = jnp.zeros((24, 16), jnp.int32)
    return jnp.concatenate([rows, cols, pad], axis=0).reshape(8, 128)


def _fmt_sc_body(embt_hbm, tail_hbm, idxtab_hbm, tbl_hbm, in0, in1, out0,
                 out1, idx_v, is0, is1, os0, os1):
    wid = lax.axis_index("s") * NC + lax.axis_index("c")
    pltpu.sync_copy(idxtab_hbm, idx_v)
    start = wid * 244 + jnp.minimum(wid, 4)
    count = jnp.where(wid < 4, 245, 244)
    npairs = (count + 1) // 2

    def in_copy(buf, c):
        return pltpu.make_async_copy(
            embt_hbm.at[:, pl.ds(c * 128, 128)], buf,
            is0 if buf is in0 else is1,
        )

    def out_copy(buf, c):
        return pltpu.make_async_copy(
            buf, tbl_hbm.at[pl.ds(c * 32, 32)],
            os0 if buf is out0 else os1,
        )

    in_copy(in0, start).start()
    in_copy(in1, start + 1).start()

    def pair(j, _):
        c0 = start + 2 * j
        c1 = c0 + 1
        in_copy(in0, c0).wait()

        @pl.when(j >= 1)
        def _():
            out_copy(out0, c0).wait()

        _tpose_block(in0, out0, idx_v, 8)
        out_copy(out0, c0).start()

        @pl.when(2 * j + 2 < count)
        def _():
            in_copy(in0, c0 + 2).start()

        @pl.when(2 * j + 1 < count)
        def _():
            in_copy(in1, c1).wait()

            @pl.when(j >= 1)
            def _():
                out_copy(out1, c1).wait()

            _tpose_block(in1, out1, idx_v, 8)
            out_copy(out1, c1).start()

            @pl.when(2 * j + 3 < count)
            def _():
                in_copy(in1, c1 + 2).start()

        return ()

    lax.fori_loop(0, npairs, pair, ())
    out_copy(out0, 0).wait()
    out_copy(out1, 0).wait()

    @pl.when(wid == NW - 1)
    def _tail():
        t = pltpu.make_async_copy(tail_hbm, in0.at[pl.ds(0, TAIL // 4)], is0)
        t.start()
        t.wait()
        t2 = pltpu.make_async_copy(
            in0.at[pl.ds(0, TAIL // 4)],
            tbl_hbm.at[pl.ds(NBLK * 32, TAIL // 4)], os0,
        )
        t2.start()
        t2.wait()


@functools.cache
def _fmt_sc():
    mesh = plsc.VectorSubcoreMesh(
        core_axis_name="c", subcore_axis_name="s", num_cores=NC
    )
    return pl.kernel(
        _fmt_sc_body,
        mesh=mesh,
        out_type=jax.ShapeDtypeStruct((V // 4, 128), jnp.float32),
        scratch_types=[
            pltpu.VMEM((32, 128), jnp.float32),
            pltpu.VMEM((32, 128), jnp.float32),
            pltpu.VMEM((32, 128), jnp.float32),
            pltpu.VMEM((32, 128), jnp.float32),
            pltpu.VMEM((8, 128), jnp.int32),
            pltpu.SemaphoreType.DMA,
            pltpu.SemaphoreType.DMA,
            pltpu.SemaphoreType.DMA,
            pltpu.SemaphoreType.DMA,
        ],
        compiler_params=pltpu.CompilerParams(
            use_tc_tiling_on_sc=True, needs_layout_passes=False
        ),
    )


GBUF = 8


def _gather_sc_body(idx_hbm, dst_hbm, table_hbm, out_hbm,
                    idx_v, dst_v, rows_v, gsem, wsem):
    wid = lax.axis_index("s") * NC + lax.axis_index("c")
    pltpu.sync_copy(idx_hbm.at[wid], idx_v)
    pltpu.sync_copy(dst_hbm.at[wid], dst_v)

    def g(b, j):
        return pltpu.make_async_copy(
            table_hbm.at[idx_v.at[j]], rows_v.at[b], gsem
        )

    def s(b, j):
        return pltpu.make_async_copy(
            rows_v.at[b], out_hbm.at[dst_v.at[j]], wsem
        )

    for b in range(GBUF):
        g(b, b).start()

    def group(q, _):
        for b in range(GBUF):
            j = q * GBUF + b
            g(b, j).wait()
            s(b, j).start()
        for b in range(GBUF):
            j = q * GBUF + b
            s(b, j).wait()

            @pl.when(j + GBUF < N_CHUNKS)
            def _():
                g(b, j + GBUF).start()

        return ()

    lax.fori_loop(0, N_CHUNKS // GBUF, group, ())


@functools.cache
def _gather_sc():
    mesh = plsc.VectorSubcoreMesh(
        core_axis_name="c", subcore_axis_name="s", num_cores=NC
    )
    return pl.kernel(
        _gather_sc_body,
        mesh=mesh,
        out_type=jax.ShapeDtypeStruct((N, E), jnp.float32),
        scratch_types=[
            pltpu.VMEM((N_CHUNKS, CHUNK), jnp.int32),
            pltpu.VMEM((N_CHUNKS, CHUNK), jnp.int32),
            pltpu.VMEM((GBUF, CHUNK, E), jnp.float32),
            pltpu.SemaphoreType.DMA,
            pltpu.SemaphoreType.DMA,
        ],
        compiler_params=pltpu.CompilerParams(use_tc_tiling_on_sc=False),
    )


def _dst_map():
    m = jnp.arange(N, dtype=jnp.int32)
    b = m // T
    t = m - b * T
    p = (t // 4 * B + b) * 4 + (t - t // 4 * 4)
    return p.reshape(NW, N_CHUNKS, CHUNK)


def _mlp_body(h_ref, w1t_ref, b1_ref, w2t_ref, b2_ref, w3_ref, b3_ref,
              o_ref, acc_ref):
    c = pl.program_id(0)

    @pl.when(c == 0)
    def _init():
        acc_ref[...] = jnp.zeros_like(acc_ref)

    acc_ref[...] += lax.dot_general(
        h_ref[0], w1t_ref[...], (((1,), (1,)), ((), ())),
        preferred_element_type=jnp.float32,
    )

    @pl.when(c == NCH - 1)
    def _finish():
        a1 = jnp.maximum(acc_ref[...] + b1_ref[...], 0.0)
        a2 = lax.dot_general(
            a1, w2t_ref[...], (((1,), (1,)), ((), ())),
            preferred_element_type=jnp.float32,
        )
        a2 = jnp.maximum(a2 + b2_ref[...], 0.0)
        a3 = jnp.dot(a2, w3_ref[...], preferred_element_type=jnp.float32)
        o_ref[...] = jax.nn.sigmoid(a3 + b3_ref[...])


def _mlp_tc(h2, W1t, b1, W2t, b2, W3, b3):
    return pl.pallas_call(
        _mlp_body,
        grid=(NCH,),
        in_specs=[
            pl.BlockSpec((1, B, 128), lambda c: (c, 0, 0)),
            pl.BlockSpec((H1, 128), lambda c: (0, c)),
            pl.BlockSpec((1, H1), lambda c: (0, 0)),
            pl.BlockSpec((H2, H1), lambda c: (0, 0)),
            pl.BlockSpec((1, H2), lambda c: (0, 0)),
            pl.BlockSpec((H2, H3), lambda c: (0, 0)),
            pl.BlockSpec((1, H3), lambda c: (0, 0)),
        ],
        out_specs=pl.BlockSpec((B, H3), lambda c: (0, 0)),
        out_shape=jax.ShapeDtypeStruct((B, H3), jnp.float32),
        scratch_shapes=[pltpu.VMEM((B, H1), jnp.float32)],
    )(h2, W1t, b1.reshape(1, H1), W2t, b2.reshape(1, H2), W3,
      b3.reshape(1, H3))


@jax.jit
def kernel(x, emb, W1, b1, W2, b2, W3, b3):
    xm = x.reshape(NW, N_CHUNKS, CHUNK)
    rows = _gather_sc()(xm, _dst_map(), emb)
    h2 = rows.reshape(NCH, B, 128)
    return _mlp_tc(h2, W1.T, b1, W2.T, b2, W3, b3)
